# baseline (speedup 1.0000x reference)
import functools

import jax
import jax.numpy as jnp
from jax import lax
from jax.experimental import pallas as pl
from jax.experimental.pallas import tpu as pltpu
from jax.experimental.pallas import tpu_sc as plsc


def _sc_embed_mean(inputs, table):
    B, L = inputs.shape
    V, D = table.shape
    assert L % 2 == 0 and D % 16 == 0
    ha = (L // 2 + 7) // 8 * 8
    hb = L - ha
    assert ha % 8 == 0 and ha <= 128 and hb <= 128

    NC, NS = 2, 16
    NW = NC * NS
    assert B % NW == 0
    b_per_w = B // NW
    CB = 32
    assert b_per_w % CB == 0
    nchunks = b_per_w // CB
    ncol = D // 16
    scale = 1.0 / L
    UN = 8
    assert L % UN == 0
    W2 = 2 * D

    table2 = table.reshape(V // 2, W2)

    mesh = plsc.VectorSubcoreMesh(core_axis_name="c", subcore_axis_name="s")

    @functools.partial(
        pl.kernel,
        mesh=mesh,
        compiler_params=pltpu.CompilerParams(use_tc_tiling_on_sc=True),
        out_type=jax.ShapeDtypeStruct((B, D), jnp.float32),
        scratch_types=[
            pltpu.VMEM((CB, L), jnp.int32),
            pltpu.VMEM((L, W2), jnp.float32),
            pltpu.VMEM((L, W2), jnp.float32),
            pltpu.VMEM((CB, D), jnp.float32),
            pltpu.VMEM((ha,), jnp.int32),
            pltpu.VMEM((hb,), jnp.int32),
            pltpu.VMEM((ha,), jnp.int32),
            pltpu.VMEM((hb,), jnp.int32),
            pltpu.SemaphoreType.DMA,
        ],
    )
    def k(inputs_hbm, table_hbm, out_hbm, idx_v, r0, r1, m_v,
          ia0, ib0, ia1, ib1, sem):
        wid = lax.axis_index("s") * NC + lax.axis_index("c")
        base = wid * b_per_w
        bufs = (r0, r1)
        ivecs = ((ia0, ib0), (ia1, ib1))

        def stage(e, ia, ib):
            offs_a = list(range(0, ha - 15, 16))
            if offs_a[-1] + 16 < ha:
                offs_a.append(ha - 16)
            for c in offs_a:
                ia[pl.ds(c, 16)] = idx_v[e, pl.ds(c, 16)] >> 1
            offs_b = list(range(0, hb - 15, 16))
            if offs_b[-1] + 16 < hb:
                offs_b.append(hb - 16)
            for c in offs_b:
                ib[pl.ds(c, 16)] = idx_v[e, pl.ds(ha + c, 16)] >> 1

        def fire(e, bi):
            ia, ib = ivecs[bi]
            stage(e, ia, ib)
            pltpu.async_copy(table_hbm.at[ia], bufs[bi].at[pl.ds(0, ha)], sem)
            pltpu.async_copy(table_hbm.at[ib], bufs[bi].at[pl.ds(ha, hb)], sem)

        def drain(bi):
            ia, ib = ivecs[bi]
            pltpu.make_async_copy(
                table_hbm.at[ia], bufs[bi].at[pl.ds(0, ha)], sem).wait()
            pltpu.make_async_copy(
                table_hbm.at[ib], bufs[bi].at[pl.ds(ha, hb)], sem).wait()

        def accum(e, buf):
            zero = jnp.zeros((16,), jnp.float32)
            acc = [zero] * ncol
            for j0 in range(0, L, 16):
                nrows = min(16, L - j0)
                p0 = min(j0, L - 16)
                pv = (idx_v[e, pl.ds(p0, 16)] & 1) * D
                offs = [pv[(j0 - p0) + u] for u in range(nrows)]
                for c in range(ncol):
                    r = [buf[j0 + u, pl.ds(offs[u] + 16 * c, 16)]
                         for u in range(nrows)]
                    while len(r) > 1:
                        r = [r[i] + r[i + 1] for i in range(0, len(r) - 1, 2)] \
                            + ([r[-1]] if len(r) % 2 else [])
                    acc[c] = acc[c] + r[0]
            for c in range(ncol):
                m_v[e, pl.ds(16 * c, 16)] = acc[c] * scale

        def chunk_body(oc, carry):
            elem0 = base + oc * CB
            pltpu.sync_copy(inputs_hbm.at[pl.ds(elem0, CB)], idx_v)
            fire(0, 0)

            def pair_body(q, carry):
                e0 = 2 * q
                for j in range(2):
                    drain(j)

                    @pl.when(e0 + j + 1 < CB)
                    def _(e=e0 + j + 1, nb=(j + 1) % 2):
                        fire(e, nb)

                    accum(e0 + j, bufs[j])
                return carry

            lax.fori_loop(0, CB // 2, pair_body, 0)
            pltpu.sync_copy(m_v, out_hbm.at[pl.ds(elem0, CB)])
            return carry

        lax.fori_loop(0, nchunks, chunk_body, 0)

    return k(inputs, table2)


def _tc_head(m, W, b):
    B, D = m.shape
    LABELS = W.shape[0]
    LP = 1024
    Wp = jnp.zeros((LP, D), jnp.float32).at[:LABELS].set(W)
    bp = jnp.full((LP, 1), -1e30, jnp.float32).at[:LABELS, 0].set(b)
    BM = 2048

    def body(m_ref, w_ref, b_ref, o_ref):
        logits = lax.dot_general(
            w_ref[...], m_ref[...], (((1,), (1,)), ((), ())),
            preferred_element_type=jnp.float32)
        logits = logits + b_ref[...]
        mx = jnp.max(logits, axis=0, keepdims=True)
        e = jnp.exp(logits - mx)
        p = e / jnp.sum(e, axis=0, keepdims=True)
        o_ref[...] = p[:LABELS, :]

    out = pl.pallas_call(
        body,
        grid=(B // BM,),
        in_specs=[
            pl.BlockSpec((BM, D), lambda i: (i, 0)),
            pl.BlockSpec((LP, D), lambda i: (0, 0)),
            pl.BlockSpec((LP, 1), lambda i: (0, 0)),
        ],
        out_specs=pl.BlockSpec((LABELS, BM), lambda i: (0, i)),
        out_shape=jax.ShapeDtypeStruct((LABELS, B), jnp.float32),
    )(m, Wp, bp)
    return out.T


def kernel(inputs, table, W, b):
    inputs = inputs.astype(jnp.int32)
    m = _sc_embed_mean(inputs, table)
    return _tc_head(m, W, b)


# speedup vs baseline: 1.0565x; 1.0565x over previous
import functools

import jax
import jax.numpy as jnp
from jax import lax
from jax.experimental import pallas as pl
from jax.experimental.pallas import tpu as pltpu
from jax.experimental.pallas import tpu_sc as plsc


def _sc_embed_mean(inputs, table):
    B, L = inputs.shape
    V, D = table.shape
    assert L % 2 == 0 and D % 16 == 0
    ha = (L // 2 + 7) // 8 * 8
    hb = L - ha
    assert ha % 8 == 0 and ha <= 128 and hb <= 128

    NC, NS = 2, 16
    NW = NC * NS
    assert B % NW == 0
    b_per_w = B // NW
    CB = 128
    assert b_per_w % CB == 0
    nchunks = b_per_w // CB
    ncol = D // 16
    scale = 1.0 / L
    UN = 8
    assert L % UN == 0
    W2 = 2 * D

    table2 = table.reshape(V // 2, W2)

    mesh = plsc.VectorSubcoreMesh(core_axis_name="c", subcore_axis_name="s")

    @functools.partial(
        pl.kernel,
        mesh=mesh,
        compiler_params=pltpu.CompilerParams(use_tc_tiling_on_sc=True),
        out_type=jax.ShapeDtypeStruct((B, D), jnp.float32),
        scratch_types=[
            pltpu.VMEM((CB, L), jnp.int32),
            pltpu.VMEM((L, W2), jnp.float32),
            pltpu.VMEM((L, W2), jnp.float32),
            pltpu.VMEM((CB, D), jnp.float32),
            pltpu.VMEM((ha,), jnp.int32),
            pltpu.VMEM((hb,), jnp.int32),
            pltpu.VMEM((ha,), jnp.int32),
            pltpu.VMEM((hb,), jnp.int32),
            pltpu.SemaphoreType.DMA,
        ],
    )
    def k(inputs_hbm, table_hbm, out_hbm, idx_v, r0, r1, m_v,
          ia0, ib0, ia1, ib1, sem):
        wid = lax.axis_index("s") * NC + lax.axis_index("c")
        base = wid * b_per_w
        bufs = (r0, r1)
        ivecs = ((ia0, ib0), (ia1, ib1))

        def stage(e, ia, ib):
            offs_a = list(range(0, ha - 15, 16))
            if offs_a[-1] + 16 < ha:
                offs_a.append(ha - 16)
            for c in offs_a:
                ia[pl.ds(c, 16)] = idx_v[e, pl.ds(c, 16)] >> 1
            offs_b = list(range(0, hb - 15, 16))
            if offs_b[-1] + 16 < hb:
                offs_b.append(hb - 16)
            for c in offs_b:
                ib[pl.ds(c, 16)] = idx_v[e, pl.ds(ha + c, 16)] >> 1

        def fire(e, bi):
            ia, ib = ivecs[bi]
            stage(e, ia, ib)
            pltpu.async_copy(table_hbm.at[ia], bufs[bi].at[pl.ds(0, ha)], sem)
            pltpu.async_copy(table_hbm.at[ib], bufs[bi].at[pl.ds(ha, hb)], sem)

        def drain(bi):
            ia, ib = ivecs[bi]
            pltpu.make_async_copy(
                table_hbm.at[ia], bufs[bi].at[pl.ds(0, ha)], sem).wait()
            pltpu.make_async_copy(
                table_hbm.at[ib], bufs[bi].at[pl.ds(ha, hb)], sem).wait()

        def accum(e, buf):
            zero = jnp.zeros((16,), jnp.float32)
            acc = [zero] * ncol
            for j0 in range(0, L, 16):
                nrows = min(16, L - j0)
                p0 = min(j0, L - 16)
                pv = (idx_v[e, pl.ds(p0, 16)] & 1) * D
                offs = [pv[(j0 - p0) + u] for u in range(nrows)]
                for c in range(ncol):
                    r = [buf[j0 + u, pl.ds(offs[u] + 16 * c, 16)]
                         for u in range(nrows)]
                    while len(r) > 1:
                        r = [r[i] + r[i + 1] for i in range(0, len(r) - 1, 2)] \
                            + ([r[-1]] if len(r) % 2 else [])
                    acc[c] = acc[c] + r[0]
            for c in range(ncol):
                m_v[e, pl.ds(16 * c, 16)] = acc[c] * scale

        def chunk_body(oc, carry):
            elem0 = base + oc * CB
            pltpu.sync_copy(inputs_hbm.at[pl.ds(elem0, CB)], idx_v)
            fire(0, 0)

            def pair_body(q, carry):
                e0 = 2 * q
                for j in range(2):
                    drain(j)

                    @pl.when(e0 + j + 1 < CB)
                    def _(e=e0 + j + 1, nb=(j + 1) % 2):
                        fire(e, nb)

                    accum(e0 + j, bufs[j])
                return carry

            lax.fori_loop(0, CB // 2, pair_body, 0)
            pltpu.sync_copy(m_v, out_hbm.at[pl.ds(elem0, CB)])
            return carry

        lax.fori_loop(0, nchunks, chunk_body, 0)

    return k(inputs, table2)


def _tc_head(m, W, b):
    B, D = m.shape
    LABELS = W.shape[0]
    LP = 1024
    Wp = jnp.zeros((LP, D), jnp.float32).at[:LABELS].set(W)
    bp = jnp.full((LP, 1), -1e30, jnp.float32).at[:LABELS, 0].set(b)
    BM = 2048

    def body(m_ref, w_ref, b_ref, o_ref):
        logits = lax.dot_general(
            w_ref[...], m_ref[...], (((1,), (1,)), ((), ())),
            preferred_element_type=jnp.float32)
        logits = logits + b_ref[...]
        mx = jnp.max(logits, axis=0, keepdims=True)
        e = jnp.exp(logits - mx)
        p = e / jnp.sum(e, axis=0, keepdims=True)
        o_ref[...] = p[:LABELS, :]

    out = pl.pallas_call(
        body,
        grid=(B // BM,),
        in_specs=[
            pl.BlockSpec((BM, D), lambda i: (i, 0)),
            pl.BlockSpec((LP, D), lambda i: (0, 0)),
            pl.BlockSpec((LP, 1), lambda i: (0, 0)),
        ],
        out_specs=pl.BlockSpec((LABELS, BM), lambda i: (0, i)),
        out_shape=jax.ShapeDtypeStruct((LABELS, B), jnp.float32),
    )(m, Wp, bp)
    return out.T


def kernel(inputs, table, W, b):
    inputs = inputs.astype(jnp.int32)
    m = _sc_embed_mean(inputs, table)
    return _tc_head(m, W, b)


# speedup vs baseline: 1.0607x; 1.0039x over previous
import functools

import jax
import jax.numpy as jnp
from jax import lax
from jax.experimental import pallas as pl
from jax.experimental.pallas import tpu as pltpu
from jax.experimental.pallas import tpu_sc as plsc


def _sc_embed_mean(inputs, table):
    B, L = inputs.shape
    V, D = table.shape
    assert L % 2 == 0 and D % 16 == 0
    ha = (L // 2 + 7) // 8 * 8
    hb = L - ha
    assert ha % 8 == 0 and ha <= 128 and hb <= 128

    NC, NS = 2, 16
    NW = NC * NS
    assert B % NW == 0
    b_per_w = B // NW
    CB = 32
    assert b_per_w % CB == 0
    nchunks = b_per_w // CB
    ncol = D // 16
    scale = 1.0 / L
    UN = 8
    assert L % UN == 0
    W2 = 2 * D

    table2 = table.reshape(V // 2, W2)

    mesh = plsc.VectorSubcoreMesh(core_axis_name="c", subcore_axis_name="s")

    @functools.partial(
        pl.kernel,
        mesh=mesh,
        compiler_params=pltpu.CompilerParams(use_tc_tiling_on_sc=True),
        out_type=jax.ShapeDtypeStruct((B, D), jnp.float32),
        scratch_types=[
            pltpu.VMEM((CB, L), jnp.int32),
            pltpu.VMEM((L, W2), jnp.float32),
            pltpu.VMEM((L, W2), jnp.float32),
            pltpu.VMEM((L, W2), jnp.float32),
            pltpu.VMEM((L, W2), jnp.float32),
            pltpu.VMEM((CB, D), jnp.float32),
            pltpu.VMEM((ha,), jnp.int32),
            pltpu.VMEM((hb,), jnp.int32),
            pltpu.VMEM((ha,), jnp.int32),
            pltpu.VMEM((hb,), jnp.int32),
            pltpu.VMEM((ha,), jnp.int32),
            pltpu.VMEM((hb,), jnp.int32),
            pltpu.VMEM((ha,), jnp.int32),
            pltpu.VMEM((hb,), jnp.int32),
            pltpu.SemaphoreType.DMA,
        ],
    )
    def k(inputs_hbm, table_hbm, out_hbm, idx_v, r0, r1, r2, r3, m_v,
          ia0, ib0, ia1, ib1, ia2, ib2, ia3, ib3, sem):
        wid = lax.axis_index("s") * NC + lax.axis_index("c")
        base = wid * b_per_w
        bufs = (r0, r1, r2, r3)
        ivecs = ((ia0, ib0), (ia1, ib1), (ia2, ib2), (ia3, ib3))

        def stage(e, ia, ib):
            offs_a = list(range(0, ha - 15, 16))
            if offs_a[-1] + 16 < ha:
                offs_a.append(ha - 16)
            for c in offs_a:
                ia[pl.ds(c, 16)] = idx_v[e, pl.ds(c, 16)] >> 1
            offs_b = list(range(0, hb - 15, 16))
            if offs_b[-1] + 16 < hb:
                offs_b.append(hb - 16)
            for c in offs_b:
                ib[pl.ds(c, 16)] = idx_v[e, pl.ds(ha + c, 16)] >> 1

        def fire(e, bi):
            ia, ib = ivecs[bi]
            stage(e, ia, ib)
            pltpu.async_copy(table_hbm.at[ia], bufs[bi].at[pl.ds(0, ha)], sem)
            pltpu.async_copy(table_hbm.at[ib], bufs[bi].at[pl.ds(ha, hb)], sem)

        def drain(bi):
            ia, ib = ivecs[bi]
            pltpu.make_async_copy(
                table_hbm.at[ia], bufs[bi].at[pl.ds(0, ha)], sem).wait()
            pltpu.make_async_copy(
                table_hbm.at[ib], bufs[bi].at[pl.ds(ha, hb)], sem).wait()

        def accum(e, buf):
            zero = jnp.zeros((16,), jnp.float32)
            acc = [zero] * ncol
            for j0 in range(0, L, 16):
                nrows = min(16, L - j0)
                p0 = min(j0, L - 16)
                pv = (idx_v[e, pl.ds(p0, 16)] & 1) * D
                offs = [pv[(j0 - p0) + u] for u in range(nrows)]
                for c in range(ncol):
                    r = [buf[j0 + u, pl.ds(offs[u] + 16 * c, 16)]
                         for u in range(nrows)]
                    while len(r) > 1:
                        r = [r[i] + r[i + 1] for i in range(0, len(r) - 1, 2)] \
                            + ([r[-1]] if len(r) % 2 else [])
                    acc[c] = acc[c] + r[0]
            for c in range(ncol):
                m_v[e, pl.ds(16 * c, 16)] = acc[c] * scale

        def chunk_body(oc, carry):
            elem0 = base + oc * CB
            pltpu.sync_copy(inputs_hbm.at[pl.ds(elem0, CB)], idx_v)
            for j in range(3):
                fire(j, j)

            def elem_body(e, carry):
                for j in range(4):
                    @pl.when(e % 4 == j)
                    def _(j=j):
                        drain(j)

                        @pl.when(e + 3 < CB)
                        def _():
                            fire(e + 3, (j + 3) % 4)

                        accum(e, bufs[j])
                return carry

            lax.fori_loop(0, CB, elem_body, 0)
            pltpu.sync_copy(m_v, out_hbm.at[pl.ds(elem0, CB)])
            return carry

        lax.fori_loop(0, nchunks, chunk_body, 0)

    return k(inputs, table2)


def _tc_head(m, W, b):
    B, D = m.shape
    LABELS = W.shape[0]
    LP = 1024
    Wp = jnp.zeros((LP, D), jnp.float32).at[:LABELS].set(W)
    bp = jnp.full((LP, 1), -1e30, jnp.float32).at[:LABELS, 0].set(b)
    BM = 2048

    def body(m_ref, w_ref, b_ref, o_ref):
        logits = lax.dot_general(
            w_ref[...], m_ref[...], (((1,), (1,)), ((), ())),
            preferred_element_type=jnp.float32)
        logits = logits + b_ref[...]
        mx = jnp.max(logits, axis=0, keepdims=True)
        e = jnp.exp(logits - mx)
        p = e / jnp.sum(e, axis=0, keepdims=True)
        o_ref[...] = p[:LABELS, :]

    out = pl.pallas_call(
        body,
        grid=(B // BM,),
        in_specs=[
            pl.BlockSpec((BM, D), lambda i: (i, 0)),
            pl.BlockSpec((LP, D), lambda i: (0, 0)),
            pl.BlockSpec((LP, 1), lambda i: (0, 0)),
        ],
        out_specs=pl.BlockSpec((LABELS, BM), lambda i: (0, i)),
        out_shape=jax.ShapeDtypeStruct((LABELS, B), jnp.float32),
    )(m, Wp, bp)
    return out.T


def kernel(inputs, table, W, b):
    inputs = inputs.astype(jnp.int32)
    m = _sc_embed_mean(inputs, table)
    return _tc_head(m, W, b)
